# Initial kernel scaffold; baseline (speedup 1.0000x reference)
import jax
import jax.numpy as jnp
from jax.experimental import pallas as pl


def kernel(core_cust_id_input, prod_code_input, dense_input, W_cust, W_prod, W1, b1, W2, b2, W3, b3):
    raise NotImplementedError("write your pallas kernel here")


# speedup vs baseline: 1.2239x; 1.2239x over previous
import functools

import jax
import jax.numpy as jnp
from jax import lax
from jax.experimental import pallas as pl
from jax.experimental.pallas import tpu as pltpu
from jax.experimental.pallas import tpu_sc as plsc

BATCH = 16384
CUST_DIM = 18
PROD_DIM = 7
CUST_PAD = 24
PROD_PAD = 8
DENSE_DIM = 13
H0, H1 = 1024, 512

NC, NS = 2, 16
NW = NC * NS
BPW = BATCH // NW
IDX_CHUNK = 128


@functools.lru_cache(maxsize=1)
def _make_sc_gather():
    mesh = plsc.VectorSubcoreMesh(core_axis_name="c", subcore_axis_name="s")

    @functools.partial(
        pl.kernel,
        mesh=mesh,
        out_type=(
            jax.ShapeDtypeStruct((BATCH, CUST_PAD), jnp.float32),
            jax.ShapeDtypeStruct((BATCH, PROD_PAD), jnp.float32),
        ),
        scratch_types=[
            pltpu.VMEM((BPW,), jnp.int32),
            pltpu.VMEM((BPW,), jnp.int32),
            pltpu.VMEM((BPW, CUST_PAD), jnp.float32),
            pltpu.VMEM((BPW, PROD_PAD), jnp.float32),
            pltpu.SemaphoreType.DMA,
        ],
        compiler_params=pltpu.CompilerParams(use_tc_tiling_on_sc=False),
    )
    def _sc_gather(cid_hbm, pid_hbm, wc_hbm, wp_hbm, cust_out, prod_out,
                   cidx_v, pidx_v, crow_v, prow_v, sem):
        wid = lax.axis_index("s") * NC + lax.axis_index("c")
        base = wid * BPW
        pltpu.sync_copy(cid_hbm.at[pl.ds(base, BPW)], cidx_v)
        pltpu.sync_copy(pid_hbm.at[pl.ds(base, BPW)], pidx_v)
        nchunks = BPW // IDX_CHUNK
        for j in range(nchunks):
            sl = pl.ds(j * IDX_CHUNK, IDX_CHUNK)
            pltpu.async_copy(wc_hbm.at[cidx_v.at[sl]], crow_v.at[sl], sem)
        for j in range(nchunks):
            sl = pl.ds(j * IDX_CHUNK, IDX_CHUNK)
            pltpu.async_copy(wp_hbm.at[pidx_v.at[sl]], prow_v.at[sl], sem)
        for j in range(nchunks):
            sl = pl.ds(j * IDX_CHUNK, IDX_CHUNK)
            pltpu.make_async_copy(wc_hbm.at[cidx_v.at[sl]], crow_v.at[sl],
                                  sem).wait()
        for j in range(nchunks):
            sl = pl.ds(j * IDX_CHUNK, IDX_CHUNK)
            pltpu.make_async_copy(wp_hbm.at[pidx_v.at[sl]], prow_v.at[sl],
                                  sem).wait()
        pltpu.sync_copy(crow_v, cust_out.at[pl.ds(base, BPW)])
        pltpu.sync_copy(prow_v, prod_out.at[pl.ds(base, BPW)])

    return _sc_gather


def _mlp_body(cust_ref, prod_ref, dense_ref, w1a_ref, w1b_ref, w1c_ref,
              b1_ref, w2_ref, b2_ref, w3_ref, b3_ref, out_ref):
    h = (jnp.dot(cust_ref[...], w1a_ref[...], preferred_element_type=jnp.float32)
         + jnp.dot(prod_ref[...], w1b_ref[...], preferred_element_type=jnp.float32)
         + jnp.dot(dense_ref[...], w1c_ref[...], preferred_element_type=jnp.float32)
         + b1_ref[...])
    h = jnp.maximum(h, 0.0)
    h = jnp.dot(h, w2_ref[...], preferred_element_type=jnp.float32) + b2_ref[...]
    h = jnp.maximum(h, 0.0)
    o = jnp.dot(h, w3_ref[...], preferred_element_type=jnp.float32) + b3_ref[...]
    out_ref[...] = 1.0 / (1.0 + jnp.exp(-o))


_ROWS = 2048


def _tc_mlp(cust_emb, prod_emb, dense, w1a, w1b, w1c, b1, w2, b2, w3, b3):
    grid = (BATCH // _ROWS,)
    full = lambda shape: pl.BlockSpec(shape, lambda i: (0, 0))
    return pl.pallas_call(
        _mlp_body,
        grid=grid,
        in_specs=[
            pl.BlockSpec((_ROWS, CUST_PAD), lambda i: (i, 0)),
            pl.BlockSpec((_ROWS, PROD_PAD), lambda i: (i, 0)),
            pl.BlockSpec((_ROWS, DENSE_DIM), lambda i: (i, 0)),
            full((CUST_PAD, H0)),
            full((PROD_PAD, H0)),
            full((DENSE_DIM, H0)),
            full((1, H0)),
            full((H0, H1)),
            full((1, H1)),
            full((H1, 1)),
            full((1, 1)),
        ],
        out_specs=pl.BlockSpec((_ROWS, 1), lambda i: (i, 0)),
        out_shape=jax.ShapeDtypeStruct((BATCH, 1), jnp.float32),
        compiler_params=pltpu.CompilerParams(
            dimension_semantics=("arbitrary",),
        ),
    )(cust_emb, prod_emb, dense, w1a, w1b, w1c, b1, w2, b2, w3, b3)


def kernel(core_cust_id_input, prod_code_input, dense_input, W_cust, W_prod,
           W1, b1, W2, b2, W3, b3):
    wc_pad = jnp.pad(W_cust, ((0, 0), (0, CUST_PAD - CUST_DIM)))
    wp_pad = jnp.pad(W_prod, ((0, 0), (0, PROD_PAD - PROD_DIM)))
    cust_emb, prod_emb = _make_sc_gather()(core_cust_id_input, prod_code_input,
                                           wc_pad, wp_pad)
    w1a = jnp.pad(W1[:CUST_DIM], ((0, CUST_PAD - CUST_DIM), (0, 0)))
    w1b = jnp.pad(W1[CUST_DIM:CUST_DIM + PROD_DIM],
                  ((0, PROD_PAD - PROD_DIM), (0, 0)))
    w1c = W1[CUST_DIM + PROD_DIM:]
    return _tc_mlp(cust_emb, prod_emb, dense_input,
                   w1a, w1b, w1c, b1.reshape(1, H0),
                   W2, b2.reshape(1, H1), W3, b3.reshape(1, 1))


# speedup vs baseline: 1.2461x; 1.0182x over previous
import functools

import jax
import jax.numpy as jnp
from jax import lax
from jax.experimental import pallas as pl
from jax.experimental.pallas import tpu as pltpu
from jax.experimental.pallas import tpu_sc as plsc

BATCH = 16384
CUST_DIM = 18
PROD_DIM = 7
CUST_PAD = 24
PROD_PAD = 8
DENSE_DIM = 13
H0, H1 = 1024, 512

NC, NS = 2, 16
NW = NC * NS
BPW = BATCH // NW
IDX_CHUNK = 128


@functools.lru_cache(maxsize=1)
def _make_sc_gather():
    mesh = plsc.VectorSubcoreMesh(core_axis_name="c", subcore_axis_name="s")

    @functools.partial(
        pl.kernel,
        mesh=mesh,
        out_type=(
            jax.ShapeDtypeStruct((BATCH, CUST_PAD), jnp.float32),
            jax.ShapeDtypeStruct((BATCH, PROD_PAD), jnp.float32),
        ),
        scratch_types=[
            pltpu.VMEM((BPW,), jnp.int32),
            pltpu.VMEM((BPW,), jnp.int32),
            pltpu.VMEM((BPW, CUST_PAD), jnp.float32),
            pltpu.VMEM((BPW, PROD_PAD), jnp.float32),
            pltpu.SemaphoreType.DMA,
        ],
        compiler_params=pltpu.CompilerParams(use_tc_tiling_on_sc=False),
    )
    def _sc_gather(cid_hbm, pid_hbm, wc_hbm, wp_hbm, cust_out, prod_out,
                   cidx_v, pidx_v, crow_v, prow_v, sem):
        wid = lax.axis_index("s") * NC + lax.axis_index("c")
        base = wid * BPW
        pltpu.sync_copy(cid_hbm.at[pl.ds(base, BPW)], cidx_v)
        pltpu.sync_copy(pid_hbm.at[pl.ds(base, BPW)], pidx_v)
        nchunks = BPW // IDX_CHUNK
        for j in range(nchunks):
            sl = pl.ds(j * IDX_CHUNK, IDX_CHUNK)
            pltpu.async_copy(wc_hbm.at[cidx_v.at[sl]], crow_v.at[sl], sem)
        for j in range(nchunks):
            sl = pl.ds(j * IDX_CHUNK, IDX_CHUNK)
            pltpu.async_copy(wp_hbm.at[pidx_v.at[sl]], prow_v.at[sl], sem)
        for j in range(nchunks):
            sl = pl.ds(j * IDX_CHUNK, IDX_CHUNK)
            pltpu.make_async_copy(wc_hbm.at[cidx_v.at[sl]], crow_v.at[sl],
                                  sem).wait()
        for j in range(nchunks):
            sl = pl.ds(j * IDX_CHUNK, IDX_CHUNK)
            pltpu.make_async_copy(wp_hbm.at[pidx_v.at[sl]], prow_v.at[sl],
                                  sem).wait()
        pltpu.sync_copy(crow_v, cust_out.at[pl.ds(base, BPW)])
        pltpu.sync_copy(prow_v, prod_out.at[pl.ds(base, BPW)])

    return _sc_gather


CUST_VOCAB = 264055
_PAD_ROWS = 4096
_PAD_GRID = (CUST_VOCAB + _PAD_ROWS - 1) // _PAD_ROWS


def _pad_body(src_ref, dst_ref):
    dst_ref[:, CUST_DIM:] = jnp.zeros((_PAD_ROWS, CUST_PAD - CUST_DIM),
                                      jnp.float32)
    dst_ref[:, :CUST_DIM] = src_ref[...]


def _tc_pad_table(w_cust):
    return pl.pallas_call(
        _pad_body,
        grid=(_PAD_GRID,),
        in_specs=[pl.BlockSpec((_PAD_ROWS, CUST_DIM), lambda i: (i, 0))],
        out_specs=pl.BlockSpec((_PAD_ROWS, CUST_PAD), lambda i: (i, 0)),
        out_shape=jax.ShapeDtypeStruct((CUST_VOCAB, CUST_PAD), jnp.float32),
        compiler_params=pltpu.CompilerParams(
            dimension_semantics=("arbitrary",),
        ),
    )(w_cust)


def _mlp_body(cust_ref, prod_ref, dense_ref, w1a_ref, w1b_ref, w1c_ref,
              b1_ref, w2_ref, b2_ref, w3_ref, b3_ref, out_ref):
    h = (jnp.dot(cust_ref[...], w1a_ref[...], preferred_element_type=jnp.float32)
         + jnp.dot(prod_ref[...], w1b_ref[...], preferred_element_type=jnp.float32)
         + jnp.dot(dense_ref[...], w1c_ref[...], preferred_element_type=jnp.float32)
         + b1_ref[...])
    h = jnp.maximum(h, 0.0)
    h = jnp.dot(h, w2_ref[...], preferred_element_type=jnp.float32) + b2_ref[...]
    h = jnp.maximum(h, 0.0)
    o = jnp.dot(h, w3_ref[...], preferred_element_type=jnp.float32) + b3_ref[...]
    out_ref[...] = 1.0 / (1.0 + jnp.exp(-o))


_ROWS = 2048


def _tc_mlp(cust_emb, prod_emb, dense, w1a, w1b, w1c, b1, w2, b2, w3, b3):
    grid = (BATCH // _ROWS,)
    full = lambda shape: pl.BlockSpec(shape, lambda i: (0, 0))
    return pl.pallas_call(
        _mlp_body,
        grid=grid,
        in_specs=[
            pl.BlockSpec((_ROWS, CUST_PAD), lambda i: (i, 0)),
            pl.BlockSpec((_ROWS, PROD_PAD), lambda i: (i, 0)),
            pl.BlockSpec((_ROWS, DENSE_DIM), lambda i: (i, 0)),
            full((CUST_PAD, H0)),
            full((PROD_PAD, H0)),
            full((DENSE_DIM, H0)),
            full((1, H0)),
            full((H0, H1)),
            full((1, H1)),
            full((H1, 1)),
            full((1, 1)),
        ],
        out_specs=pl.BlockSpec((_ROWS, 1), lambda i: (i, 0)),
        out_shape=jax.ShapeDtypeStruct((BATCH, 1), jnp.float32),
        compiler_params=pltpu.CompilerParams(
            dimension_semantics=("arbitrary",),
        ),
    )(cust_emb, prod_emb, dense, w1a, w1b, w1c, b1, w2, b2, w3, b3)


def kernel(core_cust_id_input, prod_code_input, dense_input, W_cust, W_prod,
           W1, b1, W2, b2, W3, b3):
    wc_pad = _tc_pad_table(W_cust)
    wp_pad = jnp.pad(W_prod, ((0, 0), (0, PROD_PAD - PROD_DIM)))
    cust_emb, prod_emb = _make_sc_gather()(core_cust_id_input, prod_code_input,
                                           wc_pad, wp_pad)
    w1a = jnp.pad(W1[:CUST_DIM], ((0, CUST_PAD - CUST_DIM), (0, 0)))
    w1b = jnp.pad(W1[CUST_DIM:CUST_DIM + PROD_DIM],
                  ((0, PROD_PAD - PROD_DIM), (0, 0)))
    w1c = W1[CUST_DIM + PROD_DIM:]
    return _tc_mlp(cust_emb, prod_emb, dense_input,
                   w1a, w1b, w1c, b1.reshape(1, H0),
                   W2, b2.reshape(1, H1), W3, b3.reshape(1, 1))


# speedup vs baseline: 2.8740x; 2.3063x over previous
import functools

import jax
import jax.numpy as jnp
from jax import lax
from jax.experimental import pallas as pl
from jax.experimental.pallas import tpu as pltpu
from jax.experimental.pallas import tpu_sc as plsc

BATCH = 16384
CUST_VOCAB = 264055
CUST_DIM = 18
PROD_DIM = 7
PROD_VOCAB = 129
PROD_OH = 136
TBL_W = 128
DENSE_DIM = 13
H0, H1 = 1024, 512

NC, NS = 2, 16
NW = NC * NS
BPW = BATCH // NW
IDX_CHUNK = 128


_TP_COLS = 4096
_TP_GRID = (CUST_VOCAB + _TP_COLS - 1) // _TP_COLS


def _tpad_body(wt_ref, out_ref):
    eye = jnp.eye(CUST_DIM, TBL_W, dtype=jnp.float32)
    out_ref[...] = lax.dot_general(
        wt_ref[...], eye, (((0,), (0,)), ((), ())),
        preferred_element_type=jnp.float32)


def _tc_transpose_pad(wt):
    return pl.pallas_call(
        _tpad_body,
        grid=(_TP_GRID,),
        in_specs=[pl.BlockSpec((CUST_DIM, _TP_COLS), lambda i: (0, i))],
        out_specs=pl.BlockSpec((_TP_COLS, TBL_W), lambda i: (i, 0)),
        out_shape=jax.ShapeDtypeStruct((CUST_VOCAB, TBL_W), jnp.float32),
        compiler_params=pltpu.CompilerParams(
            dimension_semantics=("arbitrary",),
        ),
    )(wt)


@functools.lru_cache(maxsize=1)
def _make_sc_gather():
    mesh = plsc.VectorSubcoreMesh(core_axis_name="c", subcore_axis_name="s")

    @functools.partial(
        pl.kernel,
        mesh=mesh,
        out_type=jax.ShapeDtypeStruct((BATCH, TBL_W), jnp.float32),
        scratch_types=[
            pltpu.VMEM((BPW,), jnp.int32),
            pltpu.VMEM((BPW, TBL_W), jnp.float32),
            pltpu.SemaphoreType.DMA,
        ],
        compiler_params=pltpu.CompilerParams(use_tc_tiling_on_sc=False),
    )
    def _sc_gather(cid_hbm, wc_hbm, cust_out, cidx_v, crow_v, sem):
        wid = lax.axis_index("s") * NC + lax.axis_index("c")
        base = wid * BPW
        pltpu.sync_copy(cid_hbm.at[pl.ds(base, BPW)], cidx_v)
        nchunks = BPW // IDX_CHUNK
        for j in range(nchunks):
            sl = pl.ds(j * IDX_CHUNK, IDX_CHUNK)
            pltpu.async_copy(wc_hbm.at[cidx_v.at[sl]], crow_v.at[sl], sem)
        for j in range(nchunks):
            sl = pl.ds(j * IDX_CHUNK, IDX_CHUNK)
            pltpu.make_async_copy(wc_hbm.at[cidx_v.at[sl]], crow_v.at[sl],
                                  sem).wait()
        pltpu.sync_copy(crow_v, cust_out.at[pl.ds(base, BPW)])

    return _sc_gather


_ROWS = 2048
_NB = BATCH // _ROWS


def _mlp_body(cust_ref, pidx_ref, denset_ref, w1a_ref, p2_ref, w1c_ref,
              b1_ref, w2_ref, b2_ref, w3_ref, b3_ref, out_ref):
    h = jnp.dot(cust_ref[...], w1a_ref[...],
                preferred_element_type=jnp.float32)
    pidx = pidx_ref[0, 0, :]
    oh = (lax.broadcasted_iota(jnp.int32, (_ROWS, PROD_OH), 1)
          == pidx[:, None]).astype(jnp.float32)
    h = h + jnp.dot(oh, p2_ref[...], preferred_element_type=jnp.float32)
    h = h + lax.dot_general(denset_ref[...], w1c_ref[...],
                            (((0,), (0,)), ((), ())),
                            preferred_element_type=jnp.float32)
    h = jnp.maximum(h + b1_ref[...], 0.0)
    h = jnp.dot(h, w2_ref[...], preferred_element_type=jnp.float32) + b2_ref[...]
    h = jnp.maximum(h, 0.0)
    o = jnp.dot(h, w3_ref[...], preferred_element_type=jnp.float32) + b3_ref[...]
    out_ref[...] = 1.0 / (1.0 + jnp.exp(-o))


def _tc_mlp(cust_emb, pidx3, denset, w1a, p2, w1c, b1, w2, b2, w3, b3):
    full = lambda shape: pl.BlockSpec(shape, lambda i: (0, 0))
    return pl.pallas_call(
        _mlp_body,
        grid=(_NB,),
        in_specs=[
            pl.BlockSpec((_ROWS, TBL_W), lambda i: (i, 0)),
            pl.BlockSpec((1, 1, _ROWS), lambda i: (i, 0, 0)),
            pl.BlockSpec((DENSE_DIM, _ROWS), lambda i: (0, i)),
            full((TBL_W, H0)),
            full((PROD_OH, H0)),
            full((DENSE_DIM, H0)),
            full((1, H0)),
            full((H0, H1)),
            full((1, H1)),
            full((H1, 1)),
            full((1, 1)),
        ],
        out_specs=pl.BlockSpec((_ROWS, 1), lambda i: (i, 0)),
        out_shape=jax.ShapeDtypeStruct((BATCH, 1), jnp.float32),
        compiler_params=pltpu.CompilerParams(
            dimension_semantics=("arbitrary",),
        ),
    )(cust_emb, pidx3, denset, w1a, p2, w1c, b1, w2, b2, w3, b3)


def kernel(core_cust_id_input, prod_code_input, dense_input, W_cust, W_prod,
           W1, b1, W2, b2, W3, b3):
    wc_pad = _tc_transpose_pad(W_cust.T)
    cust_emb = _make_sc_gather()(core_cust_id_input, wc_pad)
    w1a = jnp.pad(W1[:CUST_DIM], ((0, TBL_W - CUST_DIM), (0, 0)))
    p2 = jnp.pad(W_prod @ W1[CUST_DIM:CUST_DIM + PROD_DIM],
                 ((0, PROD_OH - PROD_VOCAB), (0, 0)))
    w1c = W1[CUST_DIM + PROD_DIM:]
    pidx3 = prod_code_input.reshape(_NB, 1, _ROWS)
    return _tc_mlp(cust_emb, pidx3, dense_input.T,
                   w1a, p2, w1c, b1.reshape(1, H0),
                   W2, b2.reshape(1, H1), W3, b3.reshape(1, 1))
